# baseline (speedup 1.0000x reference)
import functools
import math

import jax
import jax.numpy as jnp
from jax import lax
from jax.experimental import pallas as pl
from jax.experimental.pallas import tpu as pltpu
from jax.experimental.pallas import tpu_sc as plsc

N = 10000
E = 320000
D = 128
B = 64

_INV_BN = 1.0 / math.sqrt(1.0 + 1e-5)

NC = 2
NS = 16
NW = NC * NS

EPW = E // NW
ECH = 128
NFULL = EPW // ECH
ETAIL = EPW - NFULL * ECH
RPS = 624
RPS_LAST = N - (NS - 1) * RPS

_sc_mesh = plsc.VectorSubcoreMesh(core_axis_name="c", subcore_axis_name="s")


@functools.partial(
    pl.kernel,
    out_type=jax.ShapeDtypeStruct((2 * N, D), jnp.float32),
    mesh=_sc_mesh,
    scratch_types=[
        pltpu.VMEM_SHARED((N, D), jnp.float32),
        pltpu.VMEM((2, ECH), jnp.int32),
        pltpu.VMEM((ECH, D), jnp.float32),
        pltpu.VMEM((2, ECH), jnp.int32),
        pltpu.VMEM((ECH, D), jnp.float32),
        pltpu.VMEM((2, ETAIL), jnp.int32),
        pltpu.VMEM((ETAIL, D), jnp.float32),
        pltpu.SemaphoreType.DMA,
        pltpu.SemaphoreType.DMA,
        pltpu.SemaphoreType.DMA,
        pltpu.SemaphoreType.DMA,
    ],
)
def _sc_scatter(msg_hbm, ei3_hbm, src_hbm, dst_hbm, zeros_hbm, out_hbm,
                agg_sh, idx_a, rows_a, idx_b, rows_b,
                idxt_v, rowst_v, sem_a, sem_b, sem_sa, sem_sb):
    c = lax.axis_index("c")
    s = lax.axis_index("s")
    w = s * NC + c

    @pl.when(s < NS - 1)
    def _():
        pltpu.sync_copy(zeros_hbm.at[pl.ds(0, RPS)],
                        agg_sh.at[pl.ds(s * RPS, RPS)])

    @pl.when(s == NS - 1)
    def _():
        pltpu.sync_copy(zeros_hbm,
                        agg_sh.at[pl.ds((NS - 1) * RPS, RPS_LAST)])

    plsc.subcore_barrier()

    base = w * EPW

    def idx(j, idxv):
        pltpu.sync_copy(ei3_hbm.at[w * NFULL + j], idxv)

    def start_g(idxv, rows, sem):
        pltpu.make_async_copy(msg_hbm.at[idxv.at[0]], rows, sem).start()

    def wait_g(idxv, rows, sem):
        pltpu.make_async_copy(msg_hbm.at[idxv.at[0]], rows, sem).wait()

    def start_s(idxv, rows, sem):
        pltpu.async_copy(rows, agg_sh.at[idxv.at[1]], sem, add=True)

    def wait_s(idxv, rows, sem):
        pltpu.make_async_copy(rows, agg_sh.at[idxv.at[1]], sem).wait()

    idx(0, idx_a)
    start_g(idx_a, rows_a, sem_a)
    idx(1, idx_b)
    pltpu.sync_copy(zeros_hbm.at[pl.ds(0, ECH)], rows_b)
    start_s(idx_b, rows_b, sem_sb)

    def body(g, _):
        j = 2 * g
        wait_g(idx_a, rows_a, sem_a)
        start_s(idx_a, rows_a, sem_sa)
        wait_s(idx_b, rows_b, sem_sb)
        idx(j + 1, idx_b)
        start_g(idx_b, rows_b, sem_b)
        wait_g(idx_b, rows_b, sem_b)
        start_s(idx_b, rows_b, sem_sb)
        wait_s(idx_a, rows_a, sem_sa)
        idx(j + 2, idx_a)
        start_g(idx_a, rows_a, sem_a)
        return ()

    lax.fori_loop(0, NFULL // 2 - 1, body, (), unroll=False)

    wait_g(idx_a, rows_a, sem_a)
    start_s(idx_a, rows_a, sem_sa)
    wait_s(idx_b, rows_b, sem_sb)
    idx(NFULL - 1, idx_b)
    start_g(idx_b, rows_b, sem_b)
    wait_g(idx_b, rows_b, sem_b)
    start_s(idx_b, rows_b, sem_sb)
    wait_s(idx_a, rows_a, sem_sa)
    wait_s(idx_b, rows_b, sem_sb)

    offt = base + NFULL * ECH
    pltpu.sync_copy(src_hbm.at[pl.ds(offt, ETAIL)], idxt_v.at[0])
    pltpu.sync_copy(dst_hbm.at[pl.ds(offt, ETAIL)], idxt_v.at[1])
    pltpu.async_copy(msg_hbm.at[idxt_v.at[0]], rowst_v, sem_a).wait()
    pltpu.sync_copy(rowst_v, agg_sh.at[idxt_v.at[1]], add=True)

    plsc.subcore_barrier()

    @pl.when(s < NS - 1)
    def _():
        pltpu.sync_copy(agg_sh.at[pl.ds(s * RPS, RPS)],
                        out_hbm.at[pl.ds(c * N + s * RPS, RPS)])

    @pl.when(s == NS - 1)
    def _():
        pltpu.sync_copy(agg_sh.at[pl.ds((NS - 1) * RPS, RPS_LAST)],
                        out_hbm.at[pl.ds(c * N + (NS - 1) * RPS, RPS_LAST)])


RW = 400
NACT = N // RW
RCH = 16
NRCH = RW // RCH


@functools.partial(
    pl.kernel,
    out_type=[
        jax.ShapeDtypeStruct((NACT * B * D,), jnp.float32),
        jax.ShapeDtypeStruct((NACT * B * D,), jnp.float32),
    ],
    mesh=_sc_mesh,
    scratch_types=[
        pltpu.VMEM((B * D,), jnp.float32),
        pltpu.VMEM((B * D,), jnp.float32),
        pltpu.VMEM((RCH, D), jnp.float32),
        pltpu.VMEM((RCH, D), jnp.float32),
        pltpu.VMEM((RCH,), jnp.int32),
        pltpu.VMEM((RCH, D), jnp.float32),
        pltpu.VMEM((RCH, D), jnp.float32),
        pltpu.VMEM((RCH,), jnp.int32),
        pltpu.SemaphoreType.DMA,
        pltpu.SemaphoreType.DMA,
    ],
    compiler_params=pltpu.CompilerParams(needs_layout_passes=False),
)
def _sc_readout(h_hbm, hw_hbm, ids_hbm, sum_hbm, max_hbm,
                gsum_v, gmax_v, h_a, hw_a, ids_a, h_b, hw_b, ids_b,
                sem_a, sem_b):
    c = lax.axis_index("c")
    s = lax.axis_index("s")
    w = s * NC + c

    @pl.when(w < NACT)
    def _():
        zer = jnp.zeros((16,), jnp.float32)
        ninf = jnp.full((16,), -jnp.inf, jnp.float32)

        def init(k, _):
            gsum_v[pl.ds(k * 16, 16)] = zer
            gmax_v[pl.ds(k * 16, 16)] = ninf
            return ()

        lax.fori_loop(0, B * D // 16, init, (), unroll=False)

        lanes = lax.iota(jnp.int32, 16)
        base = w * RW

        def start_ch(t, hv, hwv, idsv, sem):
            off = base + t * RCH
            pltpu.make_async_copy(h_hbm.at[pl.ds(off, RCH)], hv, sem).start()
            pltpu.make_async_copy(hw_hbm.at[pl.ds(off, RCH)], hwv, sem).start()
            pltpu.make_async_copy(ids_hbm.at[pl.ds(off, RCH)], idsv,
                                  sem).start()

        def wait_ch(hv, hwv, idsv, sem):
            pltpu.make_async_copy(h_hbm.at[pl.ds(0, RCH)], hv, sem).wait()
            pltpu.make_async_copy(hw_hbm.at[pl.ds(0, RCH)], hwv, sem).wait()
            pltpu.make_async_copy(ids_hbm.at[pl.ds(0, RCH)], idsv, sem).wait()

        def compute(hv, hwv, idsv):
            ids = idsv[...]
            for i in range(RCH):
                gidv = lax.gather(
                    ids, jnp.full((16, 1), i, jnp.int32),
                    lax.GatherDimensionNumbers(
                        offset_dims=(), collapsed_slice_dims=(0,),
                        start_index_map=(0,)),
                    (1,), mode=lax.GatherScatterMode.PROMISE_IN_BOUNDS)
                rowbase = gidv * D
                for j in range(D // 16):
                    idx = rowbase + j * 16 + lanes
                    hseg = hv[i, pl.ds(j * 16, 16)]
                    hwseg = hwv[i, pl.ds(j * 16, 16)]
                    cs = plsc.load_gather(gsum_v, [idx])
                    plsc.store_scatter(gsum_v, [idx], cs + hwseg)
                    cm = plsc.load_gather(gmax_v, [idx])
                    plsc.store_scatter(gmax_v, [idx], jnp.maximum(cm, hseg))

        start_ch(0, h_a, hw_a, ids_a, sem_a)

        def pair(g, _):
            t = 2 * g
            start_ch(t + 1, h_b, hw_b, ids_b, sem_b)
            wait_ch(h_a, hw_a, ids_a, sem_a)
            compute(h_a, hw_a, ids_a)
            start_ch(t + 2, h_a, hw_a, ids_a, sem_a)
            wait_ch(h_b, hw_b, ids_b, sem_b)
            compute(h_b, hw_b, ids_b)
            return ()

        lax.fori_loop(0, NRCH // 2, pair, (), unroll=False)

        wait_ch(h_a, hw_a, ids_a, sem_a)
        compute(h_a, hw_a, ids_a)

        pltpu.sync_copy(gsum_v, sum_hbm.at[pl.ds(w * B * D, B * D)])
        pltpu.sync_copy(gmax_v, max_hbm.at[pl.ds(w * B * D, B * D)])


RB = 1000
NGRID = N // RB


def _k1_body(x_ref, wg_ref, wr_ref, br_ref, msg_ref, res_ref):
    xb = x_ref[...]
    msg_ref[...] = jnp.dot(xb, wg_ref[...], preferred_element_type=jnp.float32)
    res_ref[...] = jnp.maximum(
        jnp.dot(xb, wr_ref[...], preferred_element_type=jnp.float32)
        + br_ref[...], 0.0)


def _layer_tail(p0, p1, res, bg_ref, gamma_ref, beta_ref):
    new = jnp.maximum(p0 + p1 + bg_ref[...], 0.0) + res
    return new * (gamma_ref[...] * _INV_BN) + beta_ref[...]


def _k2_body(p0_ref, p1_ref, res_ref, bg_ref, gamma_ref, beta_ref,
             wg_ref, wr_ref, br_ref, msg_ref, res2_ref):
    h = _layer_tail(p0_ref[...], p1_ref[...], res_ref[...],
                    bg_ref, gamma_ref, beta_ref)
    msg_ref[...] = jnp.dot(h, wg_ref[...], preferred_element_type=jnp.float32)
    res2_ref[...] = jnp.maximum(
        jnp.dot(h, wr_ref[...], preferred_element_type=jnp.float32)
        + br_ref[...], 0.0)


def _k3_body(p0_ref, p1_ref, res_ref, bg_ref, gamma_ref, beta_ref,
             wa_ref, ba_ref, h_ref, hw_ref):
    h = _layer_tail(p0_ref[...], p1_ref[...], res_ref[...],
                    bg_ref, gamma_ref, beta_ref)
    gate = jax.nn.sigmoid(
        jnp.sum(h * wa_ref[...], axis=1, keepdims=True) + ba_ref[...])
    h_ref[...] = h
    hw_ref[...] = h * gate


def _k4_body(sp_ref, mp_ref, rk_ref, wrd_ref, brd_ref,
             wc1_ref, bc1_ref, wc2_ref, bc2_ref, out_ref):
    hsum = jnp.sum(sp_ref[...], axis=0)
    hmax = jnp.max(mp_ref[...], axis=0)
    rk = jnp.maximum(
        jnp.dot(rk_ref[...], wrd_ref[...], preferred_element_type=jnp.float32)
        + brd_ref[...], 0.0)
    wc1 = wc1_ref[...]
    z = (jnp.dot(hsum, wc1[0:D], preferred_element_type=jnp.float32)
         + jnp.dot(hmax, wc1[D:2 * D], preferred_element_type=jnp.float32)
         + jnp.dot(rk, wc1[2 * D:3 * D], preferred_element_type=jnp.float32)
         + bc1_ref[...])
    z = jnp.maximum(z, 0.0)
    out_ref[...] = jnp.sum(z * wc2_ref[...], axis=1, keepdims=True) + bc2_ref[...]


def _row_spec():
    return pl.BlockSpec((RB, D), lambda i: (i, 0))


def _full_spec(shape):
    nd = len(shape)
    return pl.BlockSpec(shape, lambda i: (0,) * nd)


def kernel(x, edge_index, graph_ids, rdkit_feats, Wg1, bg1, Wr1, br1,
           gamma1, beta1, Wg2, bg2, Wr2, br2, gamma2, beta2, Wa, ba,
           Wrd, brd, Wc1, bc1, Wc2, bc2):
    zeros = jnp.zeros((RPS_LAST, D), jnp.float32)
    src = edge_index[0]
    dst = edge_index[1]
    ei3 = jnp.stack(
        [src.reshape(NW, EPW)[:, :NFULL * ECH].reshape(NW, NFULL, ECH),
         dst.reshape(NW, EPW)[:, :NFULL * ECH].reshape(NW, NFULL, ECH)],
        axis=2).reshape(NW * NFULL, 2, ECH)

    bg1r = bg1.reshape(1, D); br1r = br1.reshape(1, D)
    g1r = gamma1.reshape(1, D); be1r = beta1.reshape(1, D)
    bg2r = bg2.reshape(1, D); br2r = br2.reshape(1, D)
    g2r = gamma2.reshape(1, D); be2r = beta2.reshape(1, D)
    war = Wa.reshape(1, D); bar = ba.reshape(1, 1)
    brdr = brd.reshape(1, D); bc1r = bc1.reshape(1, 256)
    wc2r = Wc2.reshape(1, 256); bc2r = bc2.reshape(1, 1)
    rdk = jnp.pad(rdkit_feats, ((0, 0), (0, 56)))
    wrdp = jnp.pad(Wrd, ((0, 56), (0, 0)))

    msg1, res1 = pl.pallas_call(
        _k1_body,
        grid=(NGRID,),
        in_specs=[_row_spec(), _full_spec((D, D)), _full_spec((D, D)),
                  _full_spec((1, D))],
        out_specs=[_row_spec(), _row_spec()],
        out_shape=[jax.ShapeDtypeStruct((N, D), jnp.float32)] * 2,
    )(x, Wg1, Wr1, br1r)

    p1 = _sc_scatter(msg1, ei3, src, dst, zeros)

    msg2, res2 = pl.pallas_call(
        _k2_body,
        grid=(NGRID,),
        in_specs=[_row_spec(),
                  pl.BlockSpec((RB, D), lambda i: (i + NGRID, 0)),
                  _row_spec(), _full_spec((1, D)), _full_spec((1, D)),
                  _full_spec((1, D)), _full_spec((D, D)), _full_spec((D, D)),
                  _full_spec((1, D))],
        out_specs=[_row_spec(), _row_spec()],
        out_shape=[jax.ShapeDtypeStruct((N, D), jnp.float32)] * 2,
    )(p1, p1, res1, bg1r, g1r, be1r, Wg2, Wr2, br2r)

    p2 = _sc_scatter(msg2, ei3, src, dst, zeros)

    h2, hw = pl.pallas_call(
        _k3_body,
        grid=(NGRID,),
        in_specs=[_row_spec(),
                  pl.BlockSpec((RB, D), lambda i: (i + NGRID, 0)),
                  _row_spec(), _full_spec((1, D)), _full_spec((1, D)),
                  _full_spec((1, D)), _full_spec((1, D)),
                  _full_spec((1, 1))],
        out_specs=[_row_spec(), _row_spec()],
        out_shape=[jax.ShapeDtypeStruct((N, D), jnp.float32)] * 2,
    )(p2, p2, res2, bg2r, g2r, be2r, war, bar)

    sums, maxs = _sc_readout(h2, hw, graph_ids)
    sp = sums.reshape(NACT, B, D)
    mp = maxs.reshape(NACT, B, D)

    preds = pl.pallas_call(
        _k4_body,
        grid=(1,),
        in_specs=[_full_spec((NACT, B, D)), _full_spec((NACT, B, D)),
                  _full_spec((B, 256)), _full_spec((256, D)),
                  _full_spec((1, D)), _full_spec((3 * D, 256)),
                  _full_spec((1, 256)), _full_spec((1, 256)),
                  _full_spec((1, 1))],
        out_specs=pl.BlockSpec((B, 1), lambda i: (0, 0)),
        out_shape=jax.ShapeDtypeStruct((B, 1), jnp.float32),
    )(sp, mp, rdk, wrdp, brdr, Wc1, bc1r, wc2r, bc2r)

    return preds


# speedup vs baseline: 1.3203x; 1.3203x over previous
import functools
import math

import jax
import jax.numpy as jnp
from jax import lax
from jax.experimental import pallas as pl
from jax.experimental.pallas import tpu as pltpu
from jax.experimental.pallas import tpu_sc as plsc

N = 10000
E = 320000
D = 128
B = 64

_INV_BN = 1.0 / math.sqrt(1.0 + 1e-5)

NC = 2
NS = 16
NW = NC * NS

EPW = E // NW
ECH = 128
NFULL = EPW // ECH
ETAIL = EPW - NFULL * ECH
RPS = 624
RPS_LAST = N - (NS - 1) * RPS

_sc_mesh = plsc.VectorSubcoreMesh(core_axis_name="c", subcore_axis_name="s")


@functools.partial(
    pl.kernel,
    out_type=jax.ShapeDtypeStruct((2 * N, D), jnp.float32),
    mesh=_sc_mesh,
    scratch_types=[
        pltpu.VMEM_SHARED((N, D), jnp.float32),
        pltpu.VMEM((2, ECH), jnp.int32),
        pltpu.VMEM((ECH, D), jnp.float32),
        pltpu.VMEM((2, ECH), jnp.int32),
        pltpu.VMEM((ECH, D), jnp.float32),
        pltpu.VMEM((2, ETAIL), jnp.int32),
        pltpu.VMEM((ETAIL, D), jnp.float32),
        pltpu.SemaphoreType.DMA,
        pltpu.SemaphoreType.DMA,
    ],
)
def _sc_scatter(msg_hbm, ei3_hbm, src_hbm, dst_hbm, zeros_hbm, out_hbm,
                agg_sh, idx_a, rows_a, idx_b, rows_b,
                idxt_v, rowst_v, sem_a, sem_b):
    c = lax.axis_index("c")
    s = lax.axis_index("s")
    w = s * NC + c

    @pl.when(s < NS - 1)
    def _():
        pltpu.sync_copy(zeros_hbm.at[pl.ds(0, RPS)],
                        agg_sh.at[pl.ds(s * RPS, RPS)])

    @pl.when(s == NS - 1)
    def _():
        pltpu.sync_copy(zeros_hbm,
                        agg_sh.at[pl.ds((NS - 1) * RPS, RPS_LAST)])

    plsc.subcore_barrier()

    base = w * EPW

    def idx(j, idxv):
        pltpu.sync_copy(ei3_hbm.at[w * NFULL + j], idxv)

    def start_g(idxv, rows, sem):
        pltpu.make_async_copy(msg_hbm.at[idxv.at[0]], rows, sem).start()

    def wait_g(idxv, rows, sem):
        pltpu.make_async_copy(msg_hbm.at[idxv.at[0]], rows, sem).wait()

    def scat(idxv, rows):
        pltpu.sync_copy(rows, agg_sh.at[idxv.at[1]], add=True)

    idx(0, idx_a)
    start_g(idx_a, rows_a, sem_a)

    def body(g, _):
        j = 2 * g
        idx(j + 1, idx_b)
        start_g(idx_b, rows_b, sem_b)
        wait_g(idx_a, rows_a, sem_a)
        scat(idx_a, rows_a)
        idx(j + 2, idx_a)
        start_g(idx_a, rows_a, sem_a)
        wait_g(idx_b, rows_b, sem_b)
        scat(idx_b, rows_b)
        return ()

    lax.fori_loop(0, NFULL // 2 - 1, body, (), unroll=False)

    idx(NFULL - 1, idx_b)
    start_g(idx_b, rows_b, sem_b)
    wait_g(idx_a, rows_a, sem_a)
    scat(idx_a, rows_a)
    wait_g(idx_b, rows_b, sem_b)
    scat(idx_b, rows_b)

    offt = base + NFULL * ECH
    pltpu.sync_copy(src_hbm.at[pl.ds(offt, ETAIL)], idxt_v.at[0])
    pltpu.sync_copy(dst_hbm.at[pl.ds(offt, ETAIL)], idxt_v.at[1])
    pltpu.async_copy(msg_hbm.at[idxt_v.at[0]], rowst_v, sem_a).wait()
    pltpu.sync_copy(rowst_v, agg_sh.at[idxt_v.at[1]], add=True)

    plsc.subcore_barrier()

    @pl.when(s < NS - 1)
    def _():
        pltpu.sync_copy(agg_sh.at[pl.ds(s * RPS, RPS)],
                        out_hbm.at[pl.ds(c * N + s * RPS, RPS)])

    @pl.when(s == NS - 1)
    def _():
        pltpu.sync_copy(agg_sh.at[pl.ds((NS - 1) * RPS, RPS_LAST)],
                        out_hbm.at[pl.ds(c * N + (NS - 1) * RPS, RPS_LAST)])


RW = 400
NACT = N // RW
RCH = 16
NRCH = RW // RCH


@functools.partial(
    pl.kernel,
    out_type=jax.ShapeDtypeStruct((NACT * B * D,), jnp.float32),
    mesh=_sc_mesh,
    scratch_types=[
        pltpu.VMEM((B * D,), jnp.float32),
        pltpu.VMEM((RCH, D), jnp.float32),
        pltpu.VMEM((RCH,), jnp.int32),
        pltpu.VMEM((RCH, D), jnp.float32),
        pltpu.VMEM((RCH,), jnp.int32),
        pltpu.SemaphoreType.DMA,
        pltpu.SemaphoreType.DMA,
    ],
    compiler_params=pltpu.CompilerParams(needs_layout_passes=False),
)
def _sc_readout(h_hbm, ids_hbm, max_hbm,
                gmax_v, h_a, ids_a, h_b, ids_b, sem_a, sem_b):
    c = lax.axis_index("c")
    s = lax.axis_index("s")
    w = s * NC + c

    @pl.when(w < NACT)
    def _():
        ninf = jnp.full((16,), -jnp.inf, jnp.float32)

        def init(k, _):
            gmax_v[pl.ds(k * 16, 16)] = ninf
            return ()

        lax.fori_loop(0, B * D // 16, init, (), unroll=False)

        lanes = lax.iota(jnp.int32, 16)
        base = w * RW

        def start_ch(t, hv, idsv, sem):
            off = base + t * RCH
            pltpu.make_async_copy(h_hbm.at[pl.ds(off, RCH)], hv, sem).start()
            pltpu.make_async_copy(ids_hbm.at[pl.ds(off, RCH)], idsv,
                                  sem).start()

        def wait_ch(hv, idsv, sem):
            pltpu.make_async_copy(h_hbm.at[pl.ds(0, RCH)], hv, sem).wait()
            pltpu.make_async_copy(ids_hbm.at[pl.ds(0, RCH)], idsv, sem).wait()

        def compute(hv, idsv):
            ids = idsv[...]
            for i in range(RCH):
                gidv = lax.gather(
                    ids, jnp.full((16, 1), i, jnp.int32),
                    lax.GatherDimensionNumbers(
                        offset_dims=(), collapsed_slice_dims=(0,),
                        start_index_map=(0,)),
                    (1,), mode=lax.GatherScatterMode.PROMISE_IN_BOUNDS)
                rowbase = gidv * D
                for j in range(D // 16):
                    idx = rowbase + j * 16 + lanes
                    hseg = hv[i, pl.ds(j * 16, 16)]
                    cm = plsc.load_gather(gmax_v, [idx])
                    plsc.store_scatter(gmax_v, [idx], jnp.maximum(cm, hseg))

        start_ch(0, h_a, ids_a, sem_a)

        def pair(g, _):
            t = 2 * g
            start_ch(t + 1, h_b, ids_b, sem_b)
            wait_ch(h_a, ids_a, sem_a)
            compute(h_a, ids_a)
            start_ch(t + 2, h_a, ids_a, sem_a)
            wait_ch(h_b, ids_b, sem_b)
            compute(h_b, ids_b)
            return ()

        lax.fori_loop(0, NRCH // 2, pair, (), unroll=False)

        wait_ch(h_a, ids_a, sem_a)
        compute(h_a, ids_a)

        pltpu.sync_copy(gmax_v, max_hbm.at[pl.ds(w * B * D, B * D)])


RB = 1000
NGRID = N // RB


def _k1_body(x_ref, wg_ref, wr_ref, br_ref, msg_ref, res_ref):
    xb = x_ref[...]
    msg_ref[...] = jnp.dot(xb, wg_ref[...], preferred_element_type=jnp.float32)
    res_ref[...] = jnp.maximum(
        jnp.dot(xb, wr_ref[...], preferred_element_type=jnp.float32)
        + br_ref[...], 0.0)


def _layer_tail(p0, p1, res, bg_ref, gamma_ref, beta_ref):
    new = jnp.maximum(p0 + p1 + bg_ref[...], 0.0) + res
    return new * (gamma_ref[...] * _INV_BN) + beta_ref[...]


def _k2_body(p0_ref, p1_ref, res_ref, bg_ref, gamma_ref, beta_ref,
             wg_ref, wr_ref, br_ref, msg_ref, res2_ref):
    h = _layer_tail(p0_ref[...], p1_ref[...], res_ref[...],
                    bg_ref, gamma_ref, beta_ref)
    msg_ref[...] = jnp.dot(h, wg_ref[...], preferred_element_type=jnp.float32)
    res2_ref[...] = jnp.maximum(
        jnp.dot(h, wr_ref[...], preferred_element_type=jnp.float32)
        + br_ref[...], 0.0)


def _k3_body(p0_ref, p1_ref, res_ref, bg_ref, gamma_ref, beta_ref,
             wa_ref, ba_ref, ids_ref, h_ref, hsum_ref):
    h = _layer_tail(p0_ref[...], p1_ref[...], res_ref[...],
                    bg_ref, gamma_ref, beta_ref)
    gate = jax.nn.sigmoid(
        jnp.sum(h * wa_ref[...], axis=1, keepdims=True) + ba_ref[...])
    h_ref[...] = h
    hw = h * gate
    ids = ids_ref[...].reshape(RB, 1)
    mask = (ids == lax.broadcasted_iota(jnp.int32, (1, B), 1)
            ).astype(jnp.float32)
    part = lax.dot_general(mask, hw, (((0,), (0,)), ((), ())),
                           preferred_element_type=jnp.float32)

    @pl.when(pl.program_id(0) == 0)
    def _():
        hsum_ref[...] = part

    @pl.when(pl.program_id(0) > 0)
    def _():
        hsum_ref[...] += part


def _k4_body(sp_ref, mp_ref, rk_ref, wrd_ref, brd_ref,
             wc1_ref, bc1_ref, wc2_ref, bc2_ref, out_ref):
    hsum = sp_ref[...]
    hmax = jnp.max(mp_ref[...], axis=0)
    rk = jnp.maximum(
        jnp.dot(rk_ref[...], wrd_ref[...], preferred_element_type=jnp.float32)
        + brd_ref[...], 0.0)
    wc1 = wc1_ref[...]
    z = (jnp.dot(hsum, wc1[0:D], preferred_element_type=jnp.float32)
         + jnp.dot(hmax, wc1[D:2 * D], preferred_element_type=jnp.float32)
         + jnp.dot(rk, wc1[2 * D:3 * D], preferred_element_type=jnp.float32)
         + bc1_ref[...])
    z = jnp.maximum(z, 0.0)
    out_ref[...] = jnp.sum(z * wc2_ref[...], axis=1, keepdims=True) + bc2_ref[...]


def _row_spec():
    return pl.BlockSpec((RB, D), lambda i: (i, 0))


def _full_spec(shape):
    nd = len(shape)
    return pl.BlockSpec(shape, lambda i: (0,) * nd)


def kernel(x, edge_index, graph_ids, rdkit_feats, Wg1, bg1, Wr1, br1,
           gamma1, beta1, Wg2, bg2, Wr2, br2, gamma2, beta2, Wa, ba,
           Wrd, brd, Wc1, bc1, Wc2, bc2):
    zeros = jnp.zeros((RPS_LAST, D), jnp.float32)
    src = edge_index[0]
    dst = edge_index[1]
    ei3 = jnp.stack(
        [src.reshape(NW, EPW)[:, :NFULL * ECH].reshape(NW, NFULL, ECH),
         dst.reshape(NW, EPW)[:, :NFULL * ECH].reshape(NW, NFULL, ECH)],
        axis=2).reshape(NW * NFULL, 2, ECH)

    bg1r = bg1.reshape(1, D); br1r = br1.reshape(1, D)
    g1r = gamma1.reshape(1, D); be1r = beta1.reshape(1, D)
    bg2r = bg2.reshape(1, D); br2r = br2.reshape(1, D)
    g2r = gamma2.reshape(1, D); be2r = beta2.reshape(1, D)
    war = Wa.reshape(1, D); bar = ba.reshape(1, 1)
    brdr = brd.reshape(1, D); bc1r = bc1.reshape(1, 256)
    wc2r = Wc2.reshape(1, 256); bc2r = bc2.reshape(1, 1)
    rdk = jnp.pad(rdkit_feats, ((0, 0), (0, 56)))
    wrdp = jnp.pad(Wrd, ((0, 56), (0, 0)))

    msg1, res1 = pl.pallas_call(
        _k1_body,
        grid=(NGRID,),
        in_specs=[_row_spec(), _full_spec((D, D)), _full_spec((D, D)),
                  _full_spec((1, D))],
        out_specs=[_row_spec(), _row_spec()],
        out_shape=[jax.ShapeDtypeStruct((N, D), jnp.float32)] * 2,
    )(x, Wg1, Wr1, br1r)

    p1 = _sc_scatter(msg1, ei3, src, dst, zeros)

    msg2, res2 = pl.pallas_call(
        _k2_body,
        grid=(NGRID,),
        in_specs=[_row_spec(),
                  pl.BlockSpec((RB, D), lambda i: (i + NGRID, 0)),
                  _row_spec(), _full_spec((1, D)), _full_spec((1, D)),
                  _full_spec((1, D)), _full_spec((D, D)), _full_spec((D, D)),
                  _full_spec((1, D))],
        out_specs=[_row_spec(), _row_spec()],
        out_shape=[jax.ShapeDtypeStruct((N, D), jnp.float32)] * 2,
    )(p1, p1, res1, bg1r, g1r, be1r, Wg2, Wr2, br2r)

    p2 = _sc_scatter(msg2, ei3, src, dst, zeros)

    ids3 = graph_ids.reshape(NGRID, 1, RB)
    h2, hsum = pl.pallas_call(
        _k3_body,
        grid=(NGRID,),
        in_specs=[_row_spec(),
                  pl.BlockSpec((RB, D), lambda i: (i + NGRID, 0)),
                  _row_spec(), _full_spec((1, D)), _full_spec((1, D)),
                  _full_spec((1, D)), _full_spec((1, D)),
                  _full_spec((1, 1)),
                  pl.BlockSpec((1, 1, RB), lambda i: (i, 0, 0))],
        out_specs=[_row_spec(), pl.BlockSpec((B, D), lambda i: (0, 0))],
        out_shape=[jax.ShapeDtypeStruct((N, D), jnp.float32),
                   jax.ShapeDtypeStruct((B, D), jnp.float32)],
    )(p2, p2, res2, bg2r, g2r, be2r, war, bar, ids3)

    maxs = _sc_readout(h2, graph_ids)
    mp = maxs.reshape(NACT, B, D)

    preds = pl.pallas_call(
        _k4_body,
        grid=(1,),
        in_specs=[_full_spec((B, D)), _full_spec((NACT, B, D)),
                  _full_spec((B, 256)), _full_spec((256, D)),
                  _full_spec((1, D)), _full_spec((3 * D, 256)),
                  _full_spec((1, 256)), _full_spec((1, 256)),
                  _full_spec((1, 1))],
        out_specs=pl.BlockSpec((B, 1), lambda i: (0, 0)),
        out_shape=jax.ShapeDtypeStruct((B, 1), jnp.float32),
    )(hsum, mp, rdk, wrdp, brdr, Wc1, bc1r, wc2r, bc2r)

    return preds


# speedup vs baseline: 1.3623x; 1.0318x over previous
import functools
import math

import jax
import jax.numpy as jnp
from jax import lax
from jax.experimental import pallas as pl
from jax.experimental.pallas import tpu as pltpu
from jax.experimental.pallas import tpu_sc as plsc

N = 10000
E = 320000
D = 128
B = 64

_INV_BN = 1.0 / math.sqrt(1.0 + 1e-5)

NC = 2
NS = 16
NW = NC * NS

EPW = E // NW
ECH = 128
NFULL = EPW // ECH
ETAIL = EPW - NFULL * ECH
RPS = 624
RPS_LAST = N - (NS - 1) * RPS

_sc_mesh = plsc.VectorSubcoreMesh(core_axis_name="c", subcore_axis_name="s")


@functools.partial(
    pl.kernel,
    out_type=jax.ShapeDtypeStruct((2 * N, D), jnp.float32),
    mesh=_sc_mesh,
    scratch_types=[
        pltpu.VMEM_SHARED((N, D), jnp.float32),
        pltpu.VMEM((2, ECH), jnp.int32),
        pltpu.VMEM((ECH, D), jnp.float32),
        pltpu.VMEM((2, ECH), jnp.int32),
        pltpu.VMEM((ECH, D), jnp.float32),
        pltpu.VMEM((2, ECH), jnp.int32),
        pltpu.VMEM((ECH, D), jnp.float32),
        pltpu.VMEM((2, ETAIL), jnp.int32),
        pltpu.SemaphoreType.DMA,
        pltpu.SemaphoreType.DMA,
        pltpu.SemaphoreType.DMA,
    ],
)
def _sc_scatter(msg_hbm, ei3_hbm, src_hbm, dst_hbm, zeros_hbm, out_hbm,
                agg_sh, idx_a, rows_a, idx_b, rows_b, idx_c, rows_c,
                idxt_v, sem_a, sem_b, sem_c):
    c = lax.axis_index("c")
    s = lax.axis_index("s")
    w = s * NC + c

    @pl.when(s < NS - 1)
    def _():
        pltpu.sync_copy(zeros_hbm.at[pl.ds(0, RPS)],
                        agg_sh.at[pl.ds(s * RPS, RPS)])

    @pl.when(s == NS - 1)
    def _():
        pltpu.sync_copy(zeros_hbm,
                        agg_sh.at[pl.ds((NS - 1) * RPS, RPS_LAST)])

    plsc.subcore_barrier()

    base = w * EPW

    def idx(j, idxv):
        pltpu.sync_copy(ei3_hbm.at[w * NFULL + j], idxv)

    def start_g(idxv, rows, sem):
        pltpu.make_async_copy(msg_hbm.at[idxv.at[0]], rows, sem).start()

    def wait_g(idxv, rows, sem):
        pltpu.make_async_copy(msg_hbm.at[idxv.at[0]], rows, sem).wait()

    def scat(idxv, rows):
        pltpu.sync_copy(rows, agg_sh.at[idxv.at[1]], add=True)

    idx(0, idx_a)
    start_g(idx_a, rows_a, sem_a)
    idx(1, idx_b)
    start_g(idx_b, rows_b, sem_b)

    def body(g, _):
        j = 3 * g
        idx(j + 2, idx_c)
        start_g(idx_c, rows_c, sem_c)
        wait_g(idx_a, rows_a, sem_a)
        scat(idx_a, rows_a)
        idx(j + 3, idx_a)
        start_g(idx_a, rows_a, sem_a)
        wait_g(idx_b, rows_b, sem_b)
        scat(idx_b, rows_b)
        idx(j + 4, idx_b)
        start_g(idx_b, rows_b, sem_b)
        wait_g(idx_c, rows_c, sem_c)
        scat(idx_c, rows_c)
        return ()

    lax.fori_loop(0, NFULL // 3 - 1, body, (), unroll=False)

    idx(NFULL - 1, idx_c)
    start_g(idx_c, rows_c, sem_c)
    wait_g(idx_a, rows_a, sem_a)
    scat(idx_a, rows_a)
    wait_g(idx_b, rows_b, sem_b)
    scat(idx_b, rows_b)
    wait_g(idx_c, rows_c, sem_c)
    scat(idx_c, rows_c)

    offt = base + NFULL * ECH
    pltpu.sync_copy(src_hbm.at[pl.ds(offt, ETAIL)], idxt_v.at[0])
    pltpu.sync_copy(dst_hbm.at[pl.ds(offt, ETAIL)], idxt_v.at[1])
    rowst = rows_a.at[pl.ds(0, ETAIL)]
    pltpu.async_copy(msg_hbm.at[idxt_v.at[0]], rowst, sem_a).wait()
    pltpu.sync_copy(rowst, agg_sh.at[idxt_v.at[1]], add=True)

    plsc.subcore_barrier()

    @pl.when(s < NS - 1)
    def _():
        pltpu.sync_copy(agg_sh.at[pl.ds(s * RPS, RPS)],
                        out_hbm.at[pl.ds(c * N + s * RPS, RPS)])

    @pl.when(s == NS - 1)
    def _():
        pltpu.sync_copy(agg_sh.at[pl.ds((NS - 1) * RPS, RPS_LAST)],
                        out_hbm.at[pl.ds(c * N + (NS - 1) * RPS, RPS_LAST)])


RW = 400
NACT = N // RW
RCH = 16
NRCH = RW // RCH


@functools.partial(
    pl.kernel,
    out_type=jax.ShapeDtypeStruct((NACT * B * D,), jnp.float32),
    mesh=_sc_mesh,
    scratch_types=[
        pltpu.VMEM((B * D,), jnp.float32),
        pltpu.VMEM((RCH, D), jnp.float32),
        pltpu.VMEM((RCH,), jnp.int32),
        pltpu.VMEM((RCH, D), jnp.float32),
        pltpu.VMEM((RCH,), jnp.int32),
        pltpu.SemaphoreType.DMA,
        pltpu.SemaphoreType.DMA,
    ],
    compiler_params=pltpu.CompilerParams(needs_layout_passes=False),
)
def _sc_readout(h_hbm, ids_hbm, max_hbm,
                gmax_v, h_a, ids_a, h_b, ids_b, sem_a, sem_b):
    c = lax.axis_index("c")
    s = lax.axis_index("s")
    w = s * NC + c

    @pl.when(w < NACT)
    def _():
        ninf = jnp.full((16,), -jnp.inf, jnp.float32)

        def init(k, _):
            gmax_v[pl.ds(k * 16, 16)] = ninf
            return ()

        lax.fori_loop(0, B * D // 16, init, (), unroll=False)

        lanes = lax.iota(jnp.int32, 16)
        base = w * RW

        def start_ch(t, hv, idsv, sem):
            off = base + t * RCH
            pltpu.make_async_copy(h_hbm.at[pl.ds(off, RCH)], hv, sem).start()
            pltpu.make_async_copy(ids_hbm.at[pl.ds(off, RCH)], idsv,
                                  sem).start()

        def wait_ch(hv, idsv, sem):
            pltpu.make_async_copy(h_hbm.at[pl.ds(0, RCH)], hv, sem).wait()
            pltpu.make_async_copy(ids_hbm.at[pl.ds(0, RCH)], idsv, sem).wait()

        def compute(hv, idsv):
            ids = idsv[...]
            for i in range(RCH):
                gidv = lax.gather(
                    ids, jnp.full((16, 1), i, jnp.int32),
                    lax.GatherDimensionNumbers(
                        offset_dims=(), collapsed_slice_dims=(0,),
                        start_index_map=(0,)),
                    (1,), mode=lax.GatherScatterMode.PROMISE_IN_BOUNDS)
                rowbase = gidv * D
                for j in range(D // 16):
                    idx = rowbase + j * 16 + lanes
                    hseg = hv[i, pl.ds(j * 16, 16)]
                    cm = plsc.load_gather(gmax_v, [idx])
                    plsc.store_scatter(gmax_v, [idx], jnp.maximum(cm, hseg))

        start_ch(0, h_a, ids_a, sem_a)

        def pair(g, _):
            t = 2 * g
            start_ch(t + 1, h_b, ids_b, sem_b)
            wait_ch(h_a, ids_a, sem_a)
            compute(h_a, ids_a)
            start_ch(t + 2, h_a, ids_a, sem_a)
            wait_ch(h_b, ids_b, sem_b)
            compute(h_b, ids_b)
            return ()

        lax.fori_loop(0, NRCH // 2, pair, (), unroll=False)

        wait_ch(h_a, ids_a, sem_a)
        compute(h_a, ids_a)

        pltpu.sync_copy(gmax_v, max_hbm.at[pl.ds(w * B * D, B * D)])


RB = 1000
NGRID = N // RB


def _k1_body(x_ref, wg_ref, wr_ref, br_ref, msg_ref, res_ref):
    xb = x_ref[...]
    msg_ref[...] = jnp.dot(xb, wg_ref[...], preferred_element_type=jnp.float32)
    res_ref[...] = jnp.maximum(
        jnp.dot(xb, wr_ref[...], preferred_element_type=jnp.float32)
        + br_ref[...], 0.0)


def _layer_tail(p0, p1, res, bg_ref, gamma_ref, beta_ref):
    new = jnp.maximum(p0 + p1 + bg_ref[...], 0.0) + res
    return new * (gamma_ref[...] * _INV_BN) + beta_ref[...]


def _k2_body(p0_ref, p1_ref, res_ref, bg_ref, gamma_ref, beta_ref,
             wg_ref, wr_ref, br_ref, msg_ref, res2_ref):
    h = _layer_tail(p0_ref[...], p1_ref[...], res_ref[...],
                    bg_ref, gamma_ref, beta_ref)
    msg_ref[...] = jnp.dot(h, wg_ref[...], preferred_element_type=jnp.float32)
    res2_ref[...] = jnp.maximum(
        jnp.dot(h, wr_ref[...], preferred_element_type=jnp.float32)
        + br_ref[...], 0.0)


def _k3_body(p0_ref, p1_ref, res_ref, bg_ref, gamma_ref, beta_ref,
             wa_ref, ba_ref, ids_ref, h_ref, hsum_ref):
    h = _layer_tail(p0_ref[...], p1_ref[...], res_ref[...],
                    bg_ref, gamma_ref, beta_ref)
    gate = jax.nn.sigmoid(
        jnp.sum(h * wa_ref[...], axis=1, keepdims=True) + ba_ref[...])
    h_ref[...] = h
    hw = h * gate
    ids = ids_ref[...].reshape(RB, 1)
    mask = (ids == lax.broadcasted_iota(jnp.int32, (1, B), 1)
            ).astype(jnp.float32)
    part = lax.dot_general(mask, hw, (((0,), (0,)), ((), ())),
                           preferred_element_type=jnp.float32)

    @pl.when(pl.program_id(0) == 0)
    def _():
        hsum_ref[...] = part

    @pl.when(pl.program_id(0) > 0)
    def _():
        hsum_ref[...] += part


def _k4_body(sp_ref, mp_ref, rk_ref, wrd_ref, brd_ref,
             wc1_ref, bc1_ref, wc2_ref, bc2_ref, out_ref):
    hsum = sp_ref[...]
    hmax = jnp.max(mp_ref[...], axis=0)
    rk = jnp.maximum(
        jnp.dot(rk_ref[...], wrd_ref[...], preferred_element_type=jnp.float32)
        + brd_ref[...], 0.0)
    wc1 = wc1_ref[...]
    z = (jnp.dot(hsum, wc1[0:D], preferred_element_type=jnp.float32)
         + jnp.dot(hmax, wc1[D:2 * D], preferred_element_type=jnp.float32)
         + jnp.dot(rk, wc1[2 * D:3 * D], preferred_element_type=jnp.float32)
         + bc1_ref[...])
    z = jnp.maximum(z, 0.0)
    out_ref[...] = jnp.sum(z * wc2_ref[...], axis=1, keepdims=True) + bc2_ref[...]


def _row_spec():
    return pl.BlockSpec((RB, D), lambda i: (i, 0))


def _full_spec(shape):
    nd = len(shape)
    return pl.BlockSpec(shape, lambda i: (0,) * nd)


def kernel(x, edge_index, graph_ids, rdkit_feats, Wg1, bg1, Wr1, br1,
           gamma1, beta1, Wg2, bg2, Wr2, br2, gamma2, beta2, Wa, ba,
           Wrd, brd, Wc1, bc1, Wc2, bc2):
    zeros = jnp.zeros((RPS_LAST, D), jnp.float32)
    src = edge_index[0]
    dst = edge_index[1]
    ei3 = jnp.stack(
        [src.reshape(NW, EPW)[:, :NFULL * ECH].reshape(NW, NFULL, ECH),
         dst.reshape(NW, EPW)[:, :NFULL * ECH].reshape(NW, NFULL, ECH)],
        axis=2).reshape(NW * NFULL, 2, ECH)

    bg1r = bg1.reshape(1, D); br1r = br1.reshape(1, D)
    g1r = gamma1.reshape(1, D); be1r = beta1.reshape(1, D)
    bg2r = bg2.reshape(1, D); br2r = br2.reshape(1, D)
    g2r = gamma2.reshape(1, D); be2r = beta2.reshape(1, D)
    war = Wa.reshape(1, D); bar = ba.reshape(1, 1)
    brdr = brd.reshape(1, D); bc1r = bc1.reshape(1, 256)
    wc2r = Wc2.reshape(1, 256); bc2r = bc2.reshape(1, 1)
    rdk = jnp.pad(rdkit_feats, ((0, 0), (0, 56)))
    wrdp = jnp.pad(Wrd, ((0, 56), (0, 0)))

    msg1, res1 = pl.pallas_call(
        _k1_body,
        grid=(NGRID,),
        in_specs=[_row_spec(), _full_spec((D, D)), _full_spec((D, D)),
                  _full_spec((1, D))],
        out_specs=[_row_spec(), _row_spec()],
        out_shape=[jax.ShapeDtypeStruct((N, D), jnp.float32)] * 2,
    )(x, Wg1, Wr1, br1r)

    p1 = _sc_scatter(msg1, ei3, src, dst, zeros)

    msg2, res2 = pl.pallas_call(
        _k2_body,
        grid=(NGRID,),
        in_specs=[_row_spec(),
                  pl.BlockSpec((RB, D), lambda i: (i + NGRID, 0)),
                  _row_spec(), _full_spec((1, D)), _full_spec((1, D)),
                  _full_spec((1, D)), _full_spec((D, D)), _full_spec((D, D)),
                  _full_spec((1, D))],
        out_specs=[_row_spec(), _row_spec()],
        out_shape=[jax.ShapeDtypeStruct((N, D), jnp.float32)] * 2,
    )(p1, p1, res1, bg1r, g1r, be1r, Wg2, Wr2, br2r)

    p2 = _sc_scatter(msg2, ei3, src, dst, zeros)

    ids3 = graph_ids.reshape(NGRID, 1, RB)
    h2, hsum = pl.pallas_call(
        _k3_body,
        grid=(NGRID,),
        in_specs=[_row_spec(),
                  pl.BlockSpec((RB, D), lambda i: (i + NGRID, 0)),
                  _row_spec(), _full_spec((1, D)), _full_spec((1, D)),
                  _full_spec((1, D)), _full_spec((1, D)),
                  _full_spec((1, 1)),
                  pl.BlockSpec((1, 1, RB), lambda i: (i, 0, 0))],
        out_specs=[_row_spec(), pl.BlockSpec((B, D), lambda i: (0, 0))],
        out_shape=[jax.ShapeDtypeStruct((N, D), jnp.float32),
                   jax.ShapeDtypeStruct((B, D), jnp.float32)],
    )(p2, p2, res2, bg2r, g2r, be2r, war, bar, ids3)

    maxs = _sc_readout(h2, graph_ids)
    mp = maxs.reshape(NACT, B, D)

    preds = pl.pallas_call(
        _k4_body,
        grid=(1,),
        in_specs=[_full_spec((B, D)), _full_spec((NACT, B, D)),
                  _full_spec((B, 256)), _full_spec((256, D)),
                  _full_spec((1, D)), _full_spec((3 * D, 256)),
                  _full_spec((1, 256)), _full_spec((1, 256)),
                  _full_spec((1, 1))],
        out_specs=pl.BlockSpec((B, 1), lambda i: (0, 0)),
        out_shape=jax.ShapeDtypeStruct((B, 1), jnp.float32),
    )(hsum, mp, rdk, wrdp, brdr, Wc1, bc1r, wc2r, bc2r)

    return preds


# speedup vs baseline: 1.4141x; 1.0381x over previous
import functools
import math

import jax
import jax.numpy as jnp
from jax import lax
from jax.experimental import pallas as pl
from jax.experimental.pallas import tpu as pltpu
from jax.experimental.pallas import tpu_sc as plsc

N = 10000
E = 320000
D = 128
B = 64

_INV_BN = 1.0 / math.sqrt(1.0 + 1e-5)

NC = 2
NS = 16
NW = NC * NS

EPW = E // NW
ECH = 128
NFULL = EPW // ECH
ETAIL = EPW - NFULL * ECH
RPS = 624
RPS_LAST = N - (NS - 1) * RPS

_sc_mesh = plsc.VectorSubcoreMesh(core_axis_name="c", subcore_axis_name="s")


@functools.partial(
    pl.kernel,
    out_type=jax.ShapeDtypeStruct((2 * N, D), jnp.float32),
    mesh=_sc_mesh,
    scratch_types=[
        pltpu.VMEM_SHARED((N, D), jnp.float32),
        pltpu.VMEM((2, ECH), jnp.int32),
        pltpu.VMEM((ECH, D), jnp.float32),
        pltpu.VMEM((2, ECH), jnp.int32),
        pltpu.VMEM((ECH, D), jnp.float32),
        pltpu.VMEM((2, ECH), jnp.int32),
        pltpu.VMEM((ECH, D), jnp.float32),
        pltpu.VMEM((2, ETAIL), jnp.int32),
        pltpu.SemaphoreType.DMA,
        pltpu.SemaphoreType.DMA,
        pltpu.SemaphoreType.DMA,
        pltpu.SemaphoreType.DMA,
        pltpu.SemaphoreType.DMA,
        pltpu.SemaphoreType.DMA,
    ],
)
def _sc_scatter(msg_hbm, ei3_hbm, src_hbm, dst_hbm, zeros_hbm, out_hbm,
                agg_sh, idx_a, rows_a, idx_b, rows_b, idx_c, rows_c,
                idxt_v, sem_a, sem_b, sem_c, sem_ia, sem_ib, sem_ic):
    c = lax.axis_index("c")
    s = lax.axis_index("s")
    w = s * NC + c

    @pl.when(s < NS - 1)
    def _():
        pltpu.sync_copy(zeros_hbm.at[pl.ds(0, RPS)],
                        agg_sh.at[pl.ds(s * RPS, RPS)])

    @pl.when(s == NS - 1)
    def _():
        pltpu.sync_copy(zeros_hbm,
                        agg_sh.at[pl.ds((NS - 1) * RPS, RPS_LAST)])

    plsc.subcore_barrier()

    base = w * EPW

    def idx(j, idxv):
        pltpu.sync_copy(ei3_hbm.at[w * NFULL + j], idxv)

    def start_g(idxv, rows, sem):
        pltpu.make_async_copy(msg_hbm.at[idxv.at[0]], rows, sem).start()

    def wait_g(idxv, rows, sem):
        pltpu.make_async_copy(msg_hbm.at[idxv.at[0]], rows, sem).wait()

    def scat(idxv, rows):
        pltpu.sync_copy(rows, agg_sh.at[idxv.at[1]], add=True)

    def idx_start(j, idxv, sem):
        pltpu.async_copy(ei3_hbm.at[w * NFULL + j], idxv, sem)

    def idx_wait(idxv, sem):
        pltpu.make_async_copy(ei3_hbm.at[w * NFULL], idxv, sem).wait()

    idx(0, idx_a)
    start_g(idx_a, rows_a, sem_a)
    idx(1, idx_b)
    start_g(idx_b, rows_b, sem_b)
    idx_start(2, idx_c, sem_ic)

    def body(g, _):
        j = 3 * g
        idx_wait(idx_c, sem_ic)
        start_g(idx_c, rows_c, sem_c)
        wait_g(idx_a, rows_a, sem_a)
        scat(idx_a, rows_a)
        idx_start(j + 3, idx_a, sem_ia)
        wait_g(idx_b, rows_b, sem_b)
        scat(idx_b, rows_b)
        idx_start(j + 4, idx_b, sem_ib)
        idx_wait(idx_a, sem_ia)
        start_g(idx_a, rows_a, sem_a)
        wait_g(idx_c, rows_c, sem_c)
        scat(idx_c, rows_c)
        idx_start(j + 5, idx_c, sem_ic)
        idx_wait(idx_b, sem_ib)
        start_g(idx_b, rows_b, sem_b)
        return ()

    lax.fori_loop(0, NFULL // 3 - 1, body, (), unroll=False)

    idx_wait(idx_c, sem_ic)
    start_g(idx_c, rows_c, sem_c)
    wait_g(idx_a, rows_a, sem_a)
    scat(idx_a, rows_a)
    wait_g(idx_b, rows_b, sem_b)
    scat(idx_b, rows_b)
    wait_g(idx_c, rows_c, sem_c)
    scat(idx_c, rows_c)

    offt = base + NFULL * ECH
    pltpu.sync_copy(src_hbm.at[pl.ds(offt, ETAIL)], idxt_v.at[0])
    pltpu.sync_copy(dst_hbm.at[pl.ds(offt, ETAIL)], idxt_v.at[1])
    rowst = rows_a.at[pl.ds(0, ETAIL)]
    pltpu.async_copy(msg_hbm.at[idxt_v.at[0]], rowst, sem_a).wait()
    pltpu.sync_copy(rowst, agg_sh.at[idxt_v.at[1]], add=True)

    plsc.subcore_barrier()

    @pl.when(s < NS - 1)
    def _():
        pltpu.sync_copy(agg_sh.at[pl.ds(s * RPS, RPS)],
                        out_hbm.at[pl.ds(c * N + s * RPS, RPS)])

    @pl.when(s == NS - 1)
    def _():
        pltpu.sync_copy(agg_sh.at[pl.ds((NS - 1) * RPS, RPS_LAST)],
                        out_hbm.at[pl.ds(c * N + (NS - 1) * RPS, RPS_LAST)])


RW = 400
NACT = N // RW
RCH = 16
NRCH = RW // RCH


@functools.partial(
    pl.kernel,
    out_type=jax.ShapeDtypeStruct((NACT * B * D,), jnp.float32),
    mesh=_sc_mesh,
    scratch_types=[
        pltpu.VMEM((B * D,), jnp.float32),
        pltpu.VMEM((RCH, D), jnp.float32),
        pltpu.VMEM((RCH,), jnp.int32),
        pltpu.VMEM((RCH, D), jnp.float32),
        pltpu.VMEM((RCH,), jnp.int32),
        pltpu.SemaphoreType.DMA,
        pltpu.SemaphoreType.DMA,
    ],
    compiler_params=pltpu.CompilerParams(needs_layout_passes=False),
)
def _sc_readout(h_hbm, ids_hbm, max_hbm,
                gmax_v, h_a, ids_a, h_b, ids_b, sem_a, sem_b):
    c = lax.axis_index("c")
    s = lax.axis_index("s")
    w = s * NC + c

    @pl.when(w < NACT)
    def _():
        ninf = jnp.full((16,), -jnp.inf, jnp.float32)

        def init(k, _):
            gmax_v[pl.ds(k * 16, 16)] = ninf
            return ()

        lax.fori_loop(0, B * D // 16, init, (), unroll=False)

        lanes = lax.iota(jnp.int32, 16)
        base = w * RW

        def start_ch(t, hv, idsv, sem):
            off = base + t * RCH
            pltpu.make_async_copy(h_hbm.at[pl.ds(off, RCH)], hv, sem).start()
            pltpu.make_async_copy(ids_hbm.at[pl.ds(off, RCH)], idsv,
                                  sem).start()

        def wait_ch(hv, idsv, sem):
            pltpu.make_async_copy(h_hbm.at[pl.ds(0, RCH)], hv, sem).wait()
            pltpu.make_async_copy(ids_hbm.at[pl.ds(0, RCH)], idsv, sem).wait()

        def compute(hv, idsv):
            ids = idsv[...]
            for i in range(RCH):
                gidv = lax.gather(
                    ids, jnp.full((16, 1), i, jnp.int32),
                    lax.GatherDimensionNumbers(
                        offset_dims=(), collapsed_slice_dims=(0,),
                        start_index_map=(0,)),
                    (1,), mode=lax.GatherScatterMode.PROMISE_IN_BOUNDS)
                rowbase = gidv * D
                for j in range(D // 16):
                    idx = rowbase + j * 16 + lanes
                    hseg = hv[i, pl.ds(j * 16, 16)]
                    cm = plsc.load_gather(gmax_v, [idx])
                    plsc.store_scatter(gmax_v, [idx], jnp.maximum(cm, hseg))

        start_ch(0, h_a, ids_a, sem_a)

        def pair(g, _):
            t = 2 * g
            start_ch(t + 1, h_b, ids_b, sem_b)
            wait_ch(h_a, ids_a, sem_a)
            compute(h_a, ids_a)
            start_ch(t + 2, h_a, ids_a, sem_a)
            wait_ch(h_b, ids_b, sem_b)
            compute(h_b, ids_b)
            return ()

        lax.fori_loop(0, NRCH // 2, pair, (), unroll=False)

        wait_ch(h_a, ids_a, sem_a)
        compute(h_a, ids_a)

        pltpu.sync_copy(gmax_v, max_hbm.at[pl.ds(w * B * D, B * D)])


RB = 1000
NGRID = N // RB


def _k1_body(x_ref, wg_ref, wr_ref, br_ref, msg_ref, res_ref):
    xb = x_ref[...]
    msg_ref[...] = jnp.dot(xb, wg_ref[...], preferred_element_type=jnp.float32)
    res_ref[...] = jnp.maximum(
        jnp.dot(xb, wr_ref[...], preferred_element_type=jnp.float32)
        + br_ref[...], 0.0)


def _layer_tail(p0, p1, res, bg_ref, gamma_ref, beta_ref):
    new = jnp.maximum(p0 + p1 + bg_ref[...], 0.0) + res
    return new * (gamma_ref[...] * _INV_BN) + beta_ref[...]


def _k2_body(p0_ref, p1_ref, res_ref, bg_ref, gamma_ref, beta_ref,
             wg_ref, wr_ref, br_ref, msg_ref, res2_ref):
    h = _layer_tail(p0_ref[...], p1_ref[...], res_ref[...],
                    bg_ref, gamma_ref, beta_ref)
    msg_ref[...] = jnp.dot(h, wg_ref[...], preferred_element_type=jnp.float32)
    res2_ref[...] = jnp.maximum(
        jnp.dot(h, wr_ref[...], preferred_element_type=jnp.float32)
        + br_ref[...], 0.0)


def _k3_body(p0_ref, p1_ref, res_ref, bg_ref, gamma_ref, beta_ref,
             wa_ref, ba_ref, ids_ref, h_ref, hsum_ref):
    h = _layer_tail(p0_ref[...], p1_ref[...], res_ref[...],
                    bg_ref, gamma_ref, beta_ref)
    gate = jax.nn.sigmoid(
        jnp.sum(h * wa_ref[...], axis=1, keepdims=True) + ba_ref[...])
    h_ref[...] = h
    hw = h * gate
    ids = ids_ref[...].reshape(RB, 1)
    mask = (ids == lax.broadcasted_iota(jnp.int32, (1, B), 1)
            ).astype(jnp.float32)
    part = lax.dot_general(mask, hw, (((0,), (0,)), ((), ())),
                           preferred_element_type=jnp.float32)

    @pl.when(pl.program_id(0) == 0)
    def _():
        hsum_ref[...] = part

    @pl.when(pl.program_id(0) > 0)
    def _():
        hsum_ref[...] += part


def _k4_body(sp_ref, mp_ref, rk_ref, wrd_ref, brd_ref,
             wc1_ref, bc1_ref, wc2_ref, bc2_ref, out_ref):
    hsum = sp_ref[...]
    hmax = jnp.max(mp_ref[...], axis=0)
    rk = jnp.maximum(
        jnp.dot(rk_ref[...], wrd_ref[...], preferred_element_type=jnp.float32)
        + brd_ref[...], 0.0)
    wc1 = wc1_ref[...]
    z = (jnp.dot(hsum, wc1[0:D], preferred_element_type=jnp.float32)
         + jnp.dot(hmax, wc1[D:2 * D], preferred_element_type=jnp.float32)
         + jnp.dot(rk, wc1[2 * D:3 * D], preferred_element_type=jnp.float32)
         + bc1_ref[...])
    z = jnp.maximum(z, 0.0)
    out_ref[...] = jnp.sum(z * wc2_ref[...], axis=1, keepdims=True) + bc2_ref[...]


def _row_spec():
    return pl.BlockSpec((RB, D), lambda i: (i, 0))


def _full_spec(shape):
    nd = len(shape)
    return pl.BlockSpec(shape, lambda i: (0,) * nd)


def kernel(x, edge_index, graph_ids, rdkit_feats, Wg1, bg1, Wr1, br1,
           gamma1, beta1, Wg2, bg2, Wr2, br2, gamma2, beta2, Wa, ba,
           Wrd, brd, Wc1, bc1, Wc2, bc2):
    zeros = jnp.zeros((RPS_LAST, D), jnp.float32)
    src = edge_index[0]
    dst = edge_index[1]
    ei3 = jnp.stack(
        [src.reshape(NW, EPW)[:, :NFULL * ECH].reshape(NW, NFULL, ECH),
         dst.reshape(NW, EPW)[:, :NFULL * ECH].reshape(NW, NFULL, ECH)],
        axis=2).reshape(NW * NFULL, 2, ECH)

    bg1r = bg1.reshape(1, D); br1r = br1.reshape(1, D)
    g1r = gamma1.reshape(1, D); be1r = beta1.reshape(1, D)
    bg2r = bg2.reshape(1, D); br2r = br2.reshape(1, D)
    g2r = gamma2.reshape(1, D); be2r = beta2.reshape(1, D)
    war = Wa.reshape(1, D); bar = ba.reshape(1, 1)
    brdr = brd.reshape(1, D); bc1r = bc1.reshape(1, 256)
    wc2r = Wc2.reshape(1, 256); bc2r = bc2.reshape(1, 1)
    rdk = jnp.pad(rdkit_feats, ((0, 0), (0, 56)))
    wrdp = jnp.pad(Wrd, ((0, 56), (0, 0)))

    msg1, res1 = pl.pallas_call(
        _k1_body,
        grid=(NGRID,),
        in_specs=[_row_spec(), _full_spec((D, D)), _full_spec((D, D)),
                  _full_spec((1, D))],
        out_specs=[_row_spec(), _row_spec()],
        out_shape=[jax.ShapeDtypeStruct((N, D), jnp.float32)] * 2,
    )(x, Wg1, Wr1, br1r)

    p1 = _sc_scatter(msg1, ei3, src, dst, zeros)

    msg2, res2 = pl.pallas_call(
        _k2_body,
        grid=(NGRID,),
        in_specs=[_row_spec(),
                  pl.BlockSpec((RB, D), lambda i: (i + NGRID, 0)),
                  _row_spec(), _full_spec((1, D)), _full_spec((1, D)),
                  _full_spec((1, D)), _full_spec((D, D)), _full_spec((D, D)),
                  _full_spec((1, D))],
        out_specs=[_row_spec(), _row_spec()],
        out_shape=[jax.ShapeDtypeStruct((N, D), jnp.float32)] * 2,
    )(p1, p1, res1, bg1r, g1r, be1r, Wg2, Wr2, br2r)

    p2 = _sc_scatter(msg2, ei3, src, dst, zeros)

    ids3 = graph_ids.reshape(NGRID, 1, RB)
    h2, hsum = pl.pallas_call(
        _k3_body,
        grid=(NGRID,),
        in_specs=[_row_spec(),
                  pl.BlockSpec((RB, D), lambda i: (i + NGRID, 0)),
                  _row_spec(), _full_spec((1, D)), _full_spec((1, D)),
                  _full_spec((1, D)), _full_spec((1, D)),
                  _full_spec((1, 1)),
                  pl.BlockSpec((1, 1, RB), lambda i: (i, 0, 0))],
        out_specs=[_row_spec(), pl.BlockSpec((B, D), lambda i: (0, 0))],
        out_shape=[jax.ShapeDtypeStruct((N, D), jnp.float32),
                   jax.ShapeDtypeStruct((B, D), jnp.float32)],
    )(p2, p2, res2, bg2r, g2r, be2r, war, bar, ids3)

    maxs = _sc_readout(h2, graph_ids)
    mp = maxs.reshape(NACT, B, D)

    preds = pl.pallas_call(
        _k4_body,
        grid=(1,),
        in_specs=[_full_spec((B, D)), _full_spec((NACT, B, D)),
                  _full_spec((B, 256)), _full_spec((256, D)),
                  _full_spec((1, D)), _full_spec((3 * D, 256)),
                  _full_spec((1, 256)), _full_spec((1, 256)),
                  _full_spec((1, 1))],
        out_specs=pl.BlockSpec((B, 1), lambda i: (0, 0)),
        out_shape=jax.ShapeDtypeStruct((B, 1), jnp.float32),
    )(hsum, mp, rdk, wrdp, brdr, Wc1, bc1r, wc2r, bc2r)

    return preds
